# baseline (speedup 1.0000x reference)
import functools

import jax
import jax.numpy as jnp
from jax.experimental import pallas as pl


def _batchnorm(x, g, b, eps=1e-5):
    mu = x.mean(axis=0)
    var = x.var(axis=0)
    return (x - mu) / jnp.sqrt(var + eps) * g + b


def _ln_bond_kernel(x_ref, g_ref, b_ref, o_ref):
    x = x_ref[...]
    mu = jnp.mean(x, axis=-1, keepdims=True)
    var = jnp.mean((x - mu) ** 2, axis=-1, keepdims=True)
    o_ref[...] = (x - mu) / jnp.sqrt(var + 1e-5) * g_ref[...] + b_ref[...]


def kernel(atom_fea, nbr_fea, nbr_fea_idx, bond_weights_ag_i, bond_weights_ag_j, bond_weights_bg_i, bond_weights_bg_j, fc_full_W, fc_full_b, bn1_g, bn1_b, bn2_g, bn2_b, ln_atom_g, ln_atom_b, bu0_W, bu0_b, bu1_W, bu1_b, bu2_W, bu2_b, ln_bond_g, ln_bond_b):
    Nn, Mm = nbr_fea_idx.shape
    Af = atom_fea.shape[-1]
    BNf = nbr_fea.shape[-1]
    atom_identity = atom_fea
    atom_nbr_fea = jnp.take(atom_fea, nbr_fea_idx.reshape(-1), axis=0).reshape(Nn, Mm, Af)
    center = jnp.broadcast_to(atom_fea[:, None, :], (Nn, Mm, Af))
    total_nbr_fea = jnp.concatenate([center, atom_nbr_fea, nbr_fea], axis=2)
    total_gated = total_nbr_fea * bond_weights_ag_i[..., None] * bond_weights_ag_j[..., None]
    total_gated = total_gated.sum(axis=1)
    lin = total_gated @ fc_full_W.T + fc_full_b
    lin = _batchnorm(lin, bn1_g, bn1_b)
    nbr_filter = jax.nn.sigmoid(lin[:, :Af])
    nbr_core = jax.nn.softplus(lin[:, Af:])
    atom_out = _batchnorm(nbr_filter * nbr_core, bn2_g, bn2_b)
    atom_out = jax.nn.softplus(atom_out)
    mu = (atom_out + atom_identity).mean(axis=-1, keepdims=True)
    t = atom_out + atom_identity
    var = t.var(axis=-1, keepdims=True)
    atom_out = (t - mu) / jnp.sqrt(var + 1e-5) * ln_atom_g + ln_atom_b
    bond_identity = nbr_fea
    nbr_atom_fea = jnp.take(atom_out, nbr_fea_idx.reshape(-1), axis=0).reshape(Nn, Mm, Af)
    center2 = jnp.broadcast_to(atom_out[:, None, :], (Nn, Mm, Af))
    total_fea = jnp.concatenate([center2, nbr_atom_fea, nbr_fea], axis=2)
    h = jax.nn.silu(total_fea @ bu0_W.T + bu0_b)
    h = jax.nn.silu(h @ bu1_W.T + bu1_b)
    h = h @ bu2_W.T + bu2_b
    new_nbr_fea = h * bond_weights_bg_i[..., None]
    pre_ln = (new_nbr_fea + bond_identity).reshape(Nn * Mm, BNf)

    TN = 800
    nbr_out = pl.pallas_call(
        _ln_bond_kernel,
        grid=(Nn * Mm // TN,),
        in_specs=[
            pl.BlockSpec((TN, BNf), lambda i: (i, 0)),
            pl.BlockSpec((1, BNf), lambda i: (0, 0)),
            pl.BlockSpec((1, BNf), lambda i: (0, 0)),
        ],
        out_specs=pl.BlockSpec((TN, BNf), lambda i: (i, 0)),
        out_shape=jax.ShapeDtypeStruct((Nn * Mm, BNf), jnp.float32),
    )(pre_ln, ln_bond_g.reshape(1, BNf), ln_bond_b.reshape(1, BNf))
    return (atom_out, nbr_out.reshape(Nn, Mm, BNf))


# speedup vs baseline: 1.5562x; 1.5562x over previous
import functools

import jax
import jax.numpy as jnp
from jax import lax
from jax.experimental import pallas as pl
from jax.experimental.pallas import tpu as pltpu
from jax.experimental.pallas import tpu_sc as plsc

F32 = jnp.float32
_EPS = 1e-5

_BCAST_DNUMS = lax.GatherDimensionNumbers(
    offset_dims=(), collapsed_slice_dims=(0,), start_index_map=(0,))


def _bcast16(v, lane):
    idx = jnp.full((16, 1), lane, jnp.int32)
    return lax.gather(v, idx, _BCAST_DNUMS, (1,),
                      indices_are_sorted=True, unique_indices=False,
                      mode=lax.GatherScatterMode.PROMISE_IN_BOUNDS)


def _softplus(x):
    return jnp.maximum(x, 0.0) + jnp.log1p(jnp.exp(-jnp.abs(x)))


def _silu(x):
    return x * jax.nn.sigmoid(x)


def _sc1_call(table, idx_r, agi_r, agj_r, npad, rpw, steps):
    A = table.shape[1]
    mesh = plsc.VectorSubcoreMesh(core_axis_name="c", subcore_axis_name="s")

    @functools.partial(
        pl.kernel, mesh=mesh,
        out_type=jax.ShapeDtypeStruct((npad, A), F32),
        scratch_types=[
            pltpu.VMEM((steps, 128), jnp.int32),
            pltpu.VMEM((steps, 128), F32),
            pltpu.VMEM((steps, 128), F32),
            pltpu.VMEM((128, A), F32),
            pltpu.VMEM((128, A), F32),
            pltpu.VMEM((rpw, A), F32),
            pltpu.SemaphoreType.DMA,
            pltpu.SemaphoreType.DMA,
        ],
    )
    def sc1(table_h, idx_h, agi_h, agj_h, out_h,
            idx_v, wi_v, wj_v, gb0, gb1, obuf, gs0, gs1):
        cid = lax.axis_index("c")
        sid = lax.axis_index("s")
        wid = sid * 2 + cid
        pltpu.sync_copy(idx_h.at[wid], idx_v)
        pltpu.sync_copy(agi_h.at[wid], wi_v)
        pltpu.sync_copy(agj_h.at[wid], wj_v)
        pltpu.async_copy(table_h.at[idx_v.at[0]], gb0, gs0)

        def compute_step(t, gbuf):
            for r4 in range(4):
                row = t * 4 + r4
                acc = [jnp.zeros((16,), F32) for _ in range(A // 16)]
                for g in range(2):
                    off = (r4 * 2 + g) * 16
                    wv = wi_v[t, pl.ds(off, 16)] * wj_v[t, pl.ds(off, 16)]
                    for ln in range(16):
                        wb = _bcast16(wv, ln)
                        e = r4 * 32 + g * 16 + ln
                        for c in range(A // 16):
                            acc[c] = acc[c] + wb * gbuf[e, pl.ds(c * 16, 16)]
                for c in range(A // 16):
                    obuf[row, pl.ds(c * 16, 16)] = acc[c]

        def pair(tt, carry):
            t = tt * 2

            @pl.when(t + 1 < steps)
            def _():
                pltpu.async_copy(table_h.at[idx_v.at[t + 1]], gb1, gs1)

            pltpu.make_async_copy(table_h.at[pl.ds(0, 128)], gb0, gs0).wait()
            compute_step(t, gb0)

            @pl.when(t + 2 < steps)
            def _():
                pltpu.async_copy(table_h.at[idx_v.at[t + 2]], gb0, gs0)

            pltpu.make_async_copy(table_h.at[pl.ds(0, 128)], gb1, gs1).wait()
            compute_step(t + 1, gb1)
            return carry

        lax.fori_loop(0, steps // 2, pair, 0)
        pltpu.sync_copy(obuf, out_h.at[pl.ds(wid * rpw, rpw)])

    return sc1(table, idx_r, agi_r, agj_r)


def _sc2_call(table, idx_r, n_edges_pad, steps, d_out):
    D = table.shape[1]
    epw = steps * 128
    chunk = 4
    mesh = plsc.VectorSubcoreMesh(core_axis_name="c", subcore_axis_name="s")

    @functools.partial(
        pl.kernel, mesh=mesh,
        out_type=jax.ShapeDtypeStruct((n_edges_pad, D), F32),
        scratch_types=[
            pltpu.VMEM((steps, 128), jnp.int32),
            pltpu.VMEM((chunk * 128, D), F32),
            pltpu.SemaphoreType.DMA,
        ],
    )
    def sc2(table_h, idx_h, out_h, idx_v, stage, gsem):
        cid = lax.axis_index("c")
        sid = lax.axis_index("s")
        wid = sid * 2 + cid
        pltpu.sync_copy(idx_h.at[wid], idx_v)

        def macro(mi, carry):
            t0 = mi * chunk
            for j in range(chunk):
                pltpu.async_copy(table_h.at[idx_v.at[t0 + j]],
                                 stage.at[pl.ds(j * 128, 128)], gsem)
            pltpu.make_async_copy(table_h.at[pl.ds(0, chunk * 128)],
                                  stage, gsem).wait()
            pltpu.sync_copy(
                stage, out_h.at[pl.ds(wid * epw + t0 * 128, chunk * 128)])
            return carry

        lax.fori_loop(0, steps // chunk, macro, 0)

    return sc2(table, idx_r)


def _k1_body(agi, agj, nbrf, sumw_o, p3_o):
    w = agi[...] * agj[...]
    sumw_o[...] = jnp.sum(w, axis=1, keepdims=True)
    M = w.shape[1]
    B = p3_o.shape[1]
    acc = jnp.zeros(p3_o.shape, F32)
    for m in range(M):
        acc = acc + w[:, m:m + 1] * nbrf[:, B * m:B * (m + 1)]
    p3_o[...] = acc


def _k2_body(af, g1, p3, sw, wct, wnt, wbt, bias, lin_o, s1_o, s2_o):
    x = af[...] * sw[...]
    lin = (jnp.dot(x, wct[...], preferred_element_type=F32)
           + jnp.dot(g1[...], wnt[...], preferred_element_type=F32)
           + jnp.dot(p3[...], wbt[...], preferred_element_type=F32)
           + bias[...])
    lin_o[...] = lin
    s1 = jnp.sum(lin, axis=0, keepdims=True)
    s2 = jnp.sum(lin * lin, axis=0, keepdims=True)

    @pl.when(pl.program_id(0) == 0)
    def _():
        s1_o[...] = s1
        s2_o[...] = s2

    @pl.when(pl.program_id(0) != 0)
    def _():
        s1_o[...] += s1
        s2_o[...] += s2


def _k3_body(lin, a1, c1, y_o, s1_o, s2_o):
    A = y_o.shape[1]
    ln = lin[...] * a1[...] + c1[...]
    y = jax.nn.sigmoid(ln[:, :A]) * _softplus(ln[:, A:])
    y_o[...] = y
    s1 = jnp.sum(y, axis=0, keepdims=True)
    s2 = jnp.sum(y * y, axis=0, keepdims=True)

    @pl.when(pl.program_id(0) == 0)
    def _():
        s1_o[...] = s1
        s2_o[...] = s2

    @pl.when(pl.program_id(0) != 0)
    def _():
        s1_o[...] += s1
        s2_o[...] += s2


def _k4_body(y, af, a2, c2, lng, lnb, bct, bnt, b0, ao_o, pc_o, pn_o):
    z = _softplus(y[...] * a2[...] + c2[...])
    t = z + af[...]
    mu = jnp.mean(t, axis=1, keepdims=True)
    d = t - mu
    var = jnp.mean(d * d, axis=1, keepdims=True)
    ao = d * lax.rsqrt(var + _EPS) * lng[...] + lnb[...]
    ao_o[...] = ao
    pc_o[...] = jnp.dot(ao, bct[...], preferred_element_type=F32) + b0[...]
    pn = jnp.dot(ao, bnt[...], preferred_element_type=F32)
    pn_o[...] = jnp.concatenate([pn, pn, pn, pn], axis=1)


def _k6_body(pc, g2, nbrf, bg, bbt, bu1t, bu2t, b1, b2, lng, lnb, out_o):
    TN, PD = pc.shape
    M = g2.shape[0] // TN
    B = nbrf.shape[1]
    E = TN * M
    pcb = jnp.broadcast_to(pc[...][:, None, :], (TN, M, PD)).reshape(E, PD)
    bp = jnp.dot(nbrf[...], bbt[...], preferred_element_type=F32)
    h0 = _silu(pcb + g2[:, :PD] + bp)
    h1 = _silu(jnp.dot(h0, bu1t[...], preferred_element_type=F32) + b1[...])
    h2 = jnp.dot(h1, bu2t[...], preferred_element_type=F32) + b2[...]
    v = h2 * bg[...] + nbrf[...]
    mu = jnp.mean(v, axis=1, keepdims=True)
    d = v - mu
    var = jnp.mean(d * d, axis=1, keepdims=True)
    out_o[...] = d * lax.rsqrt(var + _EPS) * lng[...] + lnb[...]


def kernel(atom_fea, nbr_fea, nbr_fea_idx, bond_weights_ag_i,
           bond_weights_ag_j, bond_weights_bg_i, bond_weights_bg_j,
           fc_full_W, fc_full_b, bn1_g, bn1_b, bn2_g, bn2_b,
           ln_atom_g, ln_atom_b, bu0_W, bu0_b, bu1_W, bu1_b, bu2_W, bu2_b,
           ln_bond_g, ln_bond_b):
    Nn, Mm = nbr_fea_idx.shape
    A = atom_fea.shape[-1]
    B = nbr_fea.shape[-1]
    NW = 32
    NPAD = -(-Nn // 256) * 256
    RPW = NPAD // NW
    ST = RPW * Mm // 128
    EPAD = NPAD * Mm

    def to_worker(x, dtype):
        xp = jnp.zeros((NPAD, Mm), dtype).at[:Nn].set(x)
        return xp.reshape(NW, ST, 128)

    idx_r = to_worker(nbr_fea_idx, jnp.int32)
    agi_r = to_worker(bond_weights_ag_i, F32)
    agj_r = to_worker(bond_weights_ag_j, F32)

    g1 = _sc1_call(atom_fea, idx_r, agi_r, agj_r, NPAD, RPW, ST)[:Nn]

    TN = 400
    grid = (Nn // TN,)
    nbrf2 = nbr_fea.reshape(Nn, Mm * B)
    sumw, p3 = pl.pallas_call(
        _k1_body,
        grid=grid,
        in_specs=[
            pl.BlockSpec((TN, Mm), lambda i: (i, 0)),
            pl.BlockSpec((TN, Mm), lambda i: (i, 0)),
            pl.BlockSpec((TN, Mm * B), lambda i: (i, 0)),
        ],
        out_specs=[
            pl.BlockSpec((TN, 1), lambda i: (i, 0)),
            pl.BlockSpec((TN, B), lambda i: (i, 0)),
        ],
        out_shape=[
            jax.ShapeDtypeStruct((Nn, 1), F32),
            jax.ShapeDtypeStruct((Nn, B), F32),
        ],
    )(bond_weights_ag_i, bond_weights_ag_j, nbrf2)

    C2 = 2 * A
    wct = fc_full_W[:, :A].T
    wnt = fc_full_W[:, A:2 * A].T
    wbt = fc_full_W[:, 2 * A:].T
    lin, l_s1, l_s2 = pl.pallas_call(
        _k2_body,
        grid=grid,
        in_specs=[
            pl.BlockSpec((TN, A), lambda i: (i, 0)),
            pl.BlockSpec((TN, A), lambda i: (i, 0)),
            pl.BlockSpec((TN, B), lambda i: (i, 0)),
            pl.BlockSpec((TN, 1), lambda i: (i, 0)),
            pl.BlockSpec((A, C2), lambda i: (0, 0)),
            pl.BlockSpec((A, C2), lambda i: (0, 0)),
            pl.BlockSpec((B, C2), lambda i: (0, 0)),
            pl.BlockSpec((1, C2), lambda i: (0, 0)),
        ],
        out_specs=[
            pl.BlockSpec((TN, C2), lambda i: (i, 0)),
            pl.BlockSpec((1, C2), lambda i: (0, 0)),
            pl.BlockSpec((1, C2), lambda i: (0, 0)),
        ],
        out_shape=[
            jax.ShapeDtypeStruct((Nn, C2), F32),
            jax.ShapeDtypeStruct((1, C2), F32),
            jax.ShapeDtypeStruct((1, C2), F32),
        ],
    )(atom_fea, g1, p3, sumw, wct, wnt, wbt, fc_full_b.reshape(1, C2))

    mu1 = l_s1 / Nn
    var1 = l_s2 / Nn - mu1 * mu1
    a1 = bn1_g.reshape(1, C2) * lax.rsqrt(var1 + _EPS)
    c1 = bn1_b.reshape(1, C2) - mu1 * a1

    y, y_s1, y_s2 = pl.pallas_call(
        _k3_body,
        grid=grid,
        in_specs=[
            pl.BlockSpec((TN, C2), lambda i: (i, 0)),
            pl.BlockSpec((1, C2), lambda i: (0, 0)),
            pl.BlockSpec((1, C2), lambda i: (0, 0)),
        ],
        out_specs=[
            pl.BlockSpec((TN, A), lambda i: (i, 0)),
            pl.BlockSpec((1, A), lambda i: (0, 0)),
            pl.BlockSpec((1, A), lambda i: (0, 0)),
        ],
        out_shape=[
            jax.ShapeDtypeStruct((Nn, A), F32),
            jax.ShapeDtypeStruct((1, A), F32),
            jax.ShapeDtypeStruct((1, A), F32),
        ],
    )(lin, a1, c1)

    mu2 = y_s1 / Nn
    var2 = y_s2 / Nn - mu2 * mu2
    a2 = bn2_g.reshape(1, A) * lax.rsqrt(var2 + _EPS)
    c2 = bn2_b.reshape(1, A) - mu2 * a2

    PD = bu0_W.shape[0]
    bct = bu0_W[:, :A].T
    bnt = bu0_W[:, A:2 * A].T
    atom_out, pc, pn = pl.pallas_call(
        _k4_body,
        grid=grid,
        in_specs=[
            pl.BlockSpec((TN, A), lambda i: (i, 0)),
            pl.BlockSpec((TN, A), lambda i: (i, 0)),
            pl.BlockSpec((1, A), lambda i: (0, 0)),
            pl.BlockSpec((1, A), lambda i: (0, 0)),
            pl.BlockSpec((1, A), lambda i: (0, 0)),
            pl.BlockSpec((1, A), lambda i: (0, 0)),
            pl.BlockSpec((A, PD), lambda i: (0, 0)),
            pl.BlockSpec((A, PD), lambda i: (0, 0)),
            pl.BlockSpec((1, PD), lambda i: (0, 0)),
        ],
        out_specs=[
            pl.BlockSpec((TN, A), lambda i: (i, 0)),
            pl.BlockSpec((TN, PD), lambda i: (i, 0)),
            pl.BlockSpec((TN, 4 * PD), lambda i: (i, 0)),
        ],
        out_shape=[
            jax.ShapeDtypeStruct((Nn, A), F32),
            jax.ShapeDtypeStruct((Nn, PD), F32),
            jax.ShapeDtypeStruct((Nn, 4 * PD), F32),
        ],
    )(y, atom_fea, a2, c2, ln_atom_g.reshape(1, A), ln_atom_b.reshape(1, A),
      bct, bnt, bu0_b.reshape(1, PD))

    g2 = _sc2_call(pn, idx_r, EPAD, ST, PD)[:Nn * Mm]
    G2W = g2.shape[1]

    TN6 = 200
    E6 = TN6 * Mm
    nbr_flat = nbr_fea.reshape(Nn * Mm, B)
    bg_flat = bond_weights_bg_i.reshape(Nn * Mm, 1)
    nbr_out = pl.pallas_call(
        _k6_body,
        grid=(Nn // TN6,),
        in_specs=[
            pl.BlockSpec((TN6, PD), lambda i: (i, 0)),
            pl.BlockSpec((E6, G2W), lambda i: (i, 0)),
            pl.BlockSpec((E6, B), lambda i: (i, 0)),
            pl.BlockSpec((E6, 1), lambda i: (i, 0)),
            pl.BlockSpec((B, PD), lambda i: (0, 0)),
            pl.BlockSpec((PD, B), lambda i: (0, 0)),
            pl.BlockSpec((B, B), lambda i: (0, 0)),
            pl.BlockSpec((1, B), lambda i: (0, 0)),
            pl.BlockSpec((1, B), lambda i: (0, 0)),
            pl.BlockSpec((1, B), lambda i: (0, 0)),
            pl.BlockSpec((1, B), lambda i: (0, 0)),
        ],
        out_specs=pl.BlockSpec((E6, B), lambda i: (i, 0)),
        out_shape=jax.ShapeDtypeStruct((Nn * Mm, B), F32),
    )(pc, g2, nbr_flat, bg_flat, bu0_W[:, 2 * A:].T, bu1_W.T, bu2_W.T,
      bu1_b.reshape(1, B), bu2_b.reshape(1, B),
      ln_bond_g.reshape(1, B), ln_bond_b.reshape(1, B))

    return (atom_out, nbr_out.reshape(Nn, Mm, B))


# speedup vs baseline: 1.8959x; 1.2182x over previous
import functools

import jax
import jax.numpy as jnp
from jax import lax
from jax.experimental import pallas as pl
from jax.experimental.pallas import tpu as pltpu
from jax.experimental.pallas import tpu_sc as plsc

F32 = jnp.float32
_EPS = 1e-5

_BCAST_DNUMS = lax.GatherDimensionNumbers(
    offset_dims=(), collapsed_slice_dims=(0,), start_index_map=(0,))


def _bcast16(v, lane):
    idx = jnp.full((16, 1), lane, jnp.int32)
    return lax.gather(v, idx, _BCAST_DNUMS, (1,),
                      indices_are_sorted=True, unique_indices=False,
                      mode=lax.GatherScatterMode.PROMISE_IN_BOUNDS)


def _softplus(x):
    return jnp.maximum(x, 0.0) + jnp.log1p(jnp.exp(-jnp.abs(x)))


def _silu(x):
    return x * jax.nn.sigmoid(x)


def _sc1_call(table, idx_r, w_r, npad, rpw, steps, ew):
    A = table.shape[1]
    rps = ew // 32
    mesh = plsc.VectorSubcoreMesh(core_axis_name="c", subcore_axis_name="s")

    @functools.partial(
        pl.kernel, mesh=mesh,
        out_type=jax.ShapeDtypeStruct((npad, A), F32),
        scratch_types=[
            pltpu.VMEM((steps, ew), jnp.int32),
            pltpu.VMEM((steps, ew), F32),
            pltpu.VMEM((ew, A), F32),
            pltpu.VMEM((ew, A), F32),
            pltpu.VMEM((ew, A), F32),
            pltpu.VMEM((ew, A), F32),
            pltpu.VMEM((rpw, A), F32),
            pltpu.SemaphoreType.DMA,
            pltpu.SemaphoreType.DMA,
            pltpu.SemaphoreType.DMA,
            pltpu.SemaphoreType.DMA,
        ],
    )
    def sc1(table_h, idx_h, w_h, out_h,
            idx_v, w_v, gb0, gb1, gb2, gb3, obuf, s0, s1, s2, s3):
        cid = lax.axis_index("c")
        sid = lax.axis_index("s")
        wid = sid * 2 + cid
        gbs = (gb0, gb1, gb2, gb3)
        sems = (s0, s1, s2, s3)
        pltpu.sync_copy(idx_h.at[wid], idx_v)
        pltpu.sync_copy(w_h.at[wid], w_v)
        for j in range(4):
            pltpu.async_copy(table_h.at[idx_v.at[j]], gbs[j], sems[j])

        def compute_step(t, gbuf):
            for r4 in range(rps):
                row = t * rps + r4
                acc = [jnp.zeros((16,), F32) for _ in range(A // 16)]
                for g in range(2):
                    wv = w_v[t, pl.ds((r4 * 2 + g) * 16, 16)]
                    for ln in range(16):
                        wb = _bcast16(wv, ln)
                        e = r4 * 32 + g * 16 + ln
                        for c in range(A // 16):
                            acc[c] = acc[c] + wb * gbuf[e, pl.ds(c * 16, 16)]
                for c in range(A // 16):
                    obuf[row, pl.ds(c * 16, 16)] = acc[c]

        def quad(q, carry):
            for j in range(4):
                t = q * 4 + j
                pltpu.make_async_copy(
                    table_h.at[pl.ds(0, ew)], gbs[j], sems[j]).wait()
                compute_step(t, gbs[j])

                @pl.when(t + 4 < steps)
                def _():
                    pltpu.async_copy(
                        table_h.at[idx_v.at[t + 4]], gbs[j], sems[j])
            return carry

        lax.fori_loop(0, steps // 4, quad, 0)
        pltpu.sync_copy(obuf, out_h.at[pl.ds(wid * rpw, rpw)])

    return sc1(table, idx_r, w_r)


def _sc2_call(table, idx_r, nrows_pad_q, steps):
    D = table.shape[1]
    epw_q = steps * 32
    mesh = plsc.VectorSubcoreMesh(core_axis_name="c", subcore_axis_name="s")

    @functools.partial(
        pl.kernel, mesh=mesh,
        out_type=jax.ShapeDtypeStruct((nrows_pad_q, 128), F32),
        scratch_types=[
            pltpu.VMEM((steps, 128), jnp.int32),
            pltpu.VMEM((128, D), F32),
            pltpu.VMEM((128, D), F32),
            pltpu.VMEM((128, D), F32),
            pltpu.VMEM((128, D), F32),
            pltpu.VMEM((32, 128), F32),
            pltpu.VMEM((32, 128), F32),
            pltpu.VMEM((32, 128), F32),
            pltpu.VMEM((32, 128), F32),
            pltpu.SemaphoreType.DMA,
            pltpu.SemaphoreType.DMA,
            pltpu.SemaphoreType.DMA,
            pltpu.SemaphoreType.DMA,
            pltpu.SemaphoreType.DMA,
            pltpu.SemaphoreType.DMA,
            pltpu.SemaphoreType.DMA,
            pltpu.SemaphoreType.DMA,
        ],
    )
    def sc2(table_h, idx_h, out_h, idx_v,
            gb0, gb1, gb2, gb3, ob0, ob1, ob2, ob3,
            gs0, gs1, gs2, gs3, os0, os1, os2, os3):
        cid = lax.axis_index("c")
        sid = lax.axis_index("s")
        wid = sid * 2 + cid
        gbs = (gb0, gb1, gb2, gb3)
        obs = (ob0, ob1, ob2, ob3)
        gsems = (gs0, gs1, gs2, gs3)
        osems = (os0, os1, os2, os3)
        pltpu.sync_copy(idx_h.at[wid], idx_v)
        for j in range(4):
            pltpu.async_copy(table_h.at[idx_v.at[j]], gbs[j], gsems[j])

        def visit(t, j):
            @pl.when(t >= 4)
            def _():
                pltpu.make_async_copy(
                    obs[j], out_h.at[pl.ds(0, 32)], osems[j]).wait()

            pltpu.make_async_copy(
                table_h.at[pl.ds(0, 128)], gbs[j], gsems[j]).wait()
            gbuf, obuf = gbs[j], obs[j]
            for r in range(32):
                for k in range(4):
                    e = r * 4 + k
                    obuf[r, pl.ds(k * 32, 16)] = gbuf[e, pl.ds(0, 16)]
                    obuf[r, pl.ds(k * 32 + 16, 16)] = gbuf[e, pl.ds(16, 16)]
            pltpu.async_copy(
                obuf, out_h.at[pl.ds(wid * epw_q + t * 32, 32)], osems[j])

            @pl.when(t + 4 < steps)
            def _():
                pltpu.async_copy(table_h.at[idx_v.at[t + 4]], gbs[j], gsems[j])

        def quad(q, carry):
            for j in range(4):
                visit(q * 4 + j, j)
            return carry

        lax.fori_loop(0, steps // 4, quad, 0)
        for j in range(4):
            pltpu.make_async_copy(
                obs[j], out_h.at[pl.ds(0, 32)], osems[j]).wait()

    return sc2(table, idx_r)


def _k1_body(agi, agj, nbrf, wag_o, sumw_o, p3_o):
    w = agi[...] * agj[...]
    wag_o[...] = w
    sumw_o[...] = jnp.sum(w, axis=1, keepdims=True)
    M = w.shape[1]
    B = p3_o.shape[1]
    acc = jnp.zeros(p3_o.shape, F32)
    for m in range(M):
        acc = acc + w[:, m:m + 1] * nbrf[:, B * m:B * (m + 1)]
    p3_o[...] = acc


def _k2_body(af, g1, p3, sw, wct, wnt, wbt, bias, lin_o, s1_o, s2_o):
    x = af[...] * sw[...]
    lin = (jnp.dot(x, wct[...], preferred_element_type=F32)
           + jnp.dot(g1[...], wnt[...], preferred_element_type=F32)
           + jnp.dot(p3[...], wbt[...], preferred_element_type=F32)
           + bias[...])
    lin_o[...] = lin
    s1 = jnp.sum(lin, axis=0, keepdims=True)
    s2 = jnp.sum(lin * lin, axis=0, keepdims=True)

    @pl.when(pl.program_id(0) == 0)
    def _():
        s1_o[...] = s1
        s2_o[...] = s2

    @pl.when(pl.program_id(0) != 0)
    def _():
        s1_o[...] += s1
        s2_o[...] += s2


def _k3_body(lin, a1, c1, y_o, s1_o, s2_o):
    A = y_o.shape[1]
    ln = lin[...] * a1[...] + c1[...]
    y = jax.nn.sigmoid(ln[:, :A]) * _softplus(ln[:, A:])
    y_o[...] = y
    s1 = jnp.sum(y, axis=0, keepdims=True)
    s2 = jnp.sum(y * y, axis=0, keepdims=True)

    @pl.when(pl.program_id(0) == 0)
    def _():
        s1_o[...] = s1
        s2_o[...] = s2

    @pl.when(pl.program_id(0) != 0)
    def _():
        s1_o[...] += s1
        s2_o[...] += s2


def _k4_body(y, af, a2, c2, lng, lnb, bct, bnt, b0, ao_o, pc_o, pn_o):
    z = _softplus(y[...] * a2[...] + c2[...])
    t = z + af[...]
    mu = jnp.mean(t, axis=1, keepdims=True)
    d = t - mu
    var = jnp.mean(d * d, axis=1, keepdims=True)
    ao = d * lax.rsqrt(var + _EPS) * lng[...] + lnb[...]
    ao_o[...] = ao
    pc_o[...] = jnp.dot(ao, bct[...], preferred_element_type=F32) + b0[...]
    pn = jnp.dot(ao, bnt[...], preferred_element_type=F32)
    pn_o[...] = jnp.concatenate([pn, pn, pn, pn], axis=1)


def _k6_body(pc, g2p, nbrp, bg4, bbt4, bu1t4, bu2t4, b14, b24, onesb, mavg,
             lng4, lnb4, out_o):
    TN = pc.shape[0]
    R = g2p.shape[0]
    pc4 = jnp.concatenate([pc[...]] * 4, axis=1)
    pcb = jnp.broadcast_to(pc4[:, None, :], (TN, 8, 128)).reshape(R, 128)
    bp = jnp.dot(nbrp[...], bbt4[...], preferred_element_type=F32)
    h0 = _silu(pcb + g2p[...] + bp)
    h1 = _silu(jnp.dot(h0, bu1t4[...], preferred_element_type=F32) + b14[...])
    h2 = jnp.dot(h1, bu2t4[...], preferred_element_type=F32) + b24[...]
    bgb = jnp.dot(bg4[...], onesb[...], preferred_element_type=F32)
    v = h2 * bgb + nbrp[...]
    mu = jnp.dot(v, mavg[...], preferred_element_type=F32)
    d = v - mu
    var = jnp.dot(d * d, mavg[...], preferred_element_type=F32)
    out_o[...] = d * lax.rsqrt(var + _EPS) * lng4[...] + lnb4[...]


def kernel(atom_fea, nbr_fea, nbr_fea_idx, bond_weights_ag_i,
           bond_weights_ag_j, bond_weights_bg_i, bond_weights_bg_j,
           fc_full_W, fc_full_b, bn1_g, bn1_b, bn2_g, bn2_b,
           ln_atom_g, ln_atom_b, bu0_W, bu0_b, bu1_W, bu1_b, bu2_W, bu2_b,
           ln_bond_g, ln_bond_b):
    Nn, Mm = nbr_fea_idx.shape
    A = atom_fea.shape[-1]
    B = nbr_fea.shape[-1]
    NW = 32
    NPAD = -(-Nn // 256) * 256
    RPW = NPAD // NW
    EW1 = 64
    ST1 = RPW * Mm // EW1
    ST2 = RPW * Mm // 128

    def to_worker(x, dtype, st, ew):
        xp = jnp.zeros((NPAD, Mm), dtype).at[:Nn].set(x)
        return xp.reshape(NW, st, ew)

    idx_r1 = to_worker(nbr_fea_idx, jnp.int32, ST1, EW1)
    idx_r2 = to_worker(nbr_fea_idx, jnp.int32, ST2, 128)

    TN = 400
    grid = (Nn // TN,)
    nbrf2 = nbr_fea.reshape(Nn, Mm * B)
    wag, sumw, p3 = pl.pallas_call(
        _k1_body,
        grid=grid,
        in_specs=[
            pl.BlockSpec((TN, Mm), lambda i: (i, 0)),
            pl.BlockSpec((TN, Mm), lambda i: (i, 0)),
            pl.BlockSpec((TN, Mm * B), lambda i: (i, 0)),
        ],
        out_specs=[
            pl.BlockSpec((TN, Mm), lambda i: (i, 0)),
            pl.BlockSpec((TN, 1), lambda i: (i, 0)),
            pl.BlockSpec((TN, B), lambda i: (i, 0)),
        ],
        out_shape=[
            jax.ShapeDtypeStruct((Nn, Mm), F32),
            jax.ShapeDtypeStruct((Nn, 1), F32),
            jax.ShapeDtypeStruct((Nn, B), F32),
        ],
    )(bond_weights_ag_i, bond_weights_ag_j, nbrf2)

    w_r = to_worker(wag, F32, ST1, EW1)

    g1 = _sc1_call(atom_fea, idx_r1, w_r, NPAD, RPW, ST1, EW1)

    C2 = 2 * A
    wct = fc_full_W[:, :A].T
    wnt = fc_full_W[:, A:2 * A].T
    wbt = fc_full_W[:, 2 * A:].T
    lin, l_s1, l_s2 = pl.pallas_call(
        _k2_body,
        grid=grid,
        in_specs=[
            pl.BlockSpec((TN, A), lambda i: (i, 0)),
            pl.BlockSpec((TN, A), lambda i: (i, 0)),
            pl.BlockSpec((TN, B), lambda i: (i, 0)),
            pl.BlockSpec((TN, 1), lambda i: (i, 0)),
            pl.BlockSpec((A, C2), lambda i: (0, 0)),
            pl.BlockSpec((A, C2), lambda i: (0, 0)),
            pl.BlockSpec((B, C2), lambda i: (0, 0)),
            pl.BlockSpec((1, C2), lambda i: (0, 0)),
        ],
        out_specs=[
            pl.BlockSpec((TN, C2), lambda i: (i, 0)),
            pl.BlockSpec((1, C2), lambda i: (0, 0)),
            pl.BlockSpec((1, C2), lambda i: (0, 0)),
        ],
        out_shape=[
            jax.ShapeDtypeStruct((Nn, C2), F32),
            jax.ShapeDtypeStruct((1, C2), F32),
            jax.ShapeDtypeStruct((1, C2), F32),
        ],
    )(atom_fea, g1, p3, sumw, wct, wnt, wbt, fc_full_b.reshape(1, C2))

    mu1 = l_s1 / Nn
    var1 = l_s2 / Nn - mu1 * mu1
    a1 = bn1_g.reshape(1, C2) * lax.rsqrt(var1 + _EPS)
    c1 = bn1_b.reshape(1, C2) - mu1 * a1

    y, y_s1, y_s2 = pl.pallas_call(
        _k3_body,
        grid=grid,
        in_specs=[
            pl.BlockSpec((TN, C2), lambda i: (i, 0)),
            pl.BlockSpec((1, C2), lambda i: (0, 0)),
            pl.BlockSpec((1, C2), lambda i: (0, 0)),
        ],
        out_specs=[
            pl.BlockSpec((TN, A), lambda i: (i, 0)),
            pl.BlockSpec((1, A), lambda i: (0, 0)),
            pl.BlockSpec((1, A), lambda i: (0, 0)),
        ],
        out_shape=[
            jax.ShapeDtypeStruct((Nn, A), F32),
            jax.ShapeDtypeStruct((1, A), F32),
            jax.ShapeDtypeStruct((1, A), F32),
        ],
    )(lin, a1, c1)

    mu2 = y_s1 / Nn
    var2 = y_s2 / Nn - mu2 * mu2
    a2 = bn2_g.reshape(1, A) * lax.rsqrt(var2 + _EPS)
    c2 = bn2_b.reshape(1, A) - mu2 * a2

    PD = bu0_W.shape[0]
    bct = bu0_W[:, :A].T
    bnt = bu0_W[:, A:2 * A].T
    atom_out, pc, pn = pl.pallas_call(
        _k4_body,
        grid=grid,
        in_specs=[
            pl.BlockSpec((TN, A), lambda i: (i, 0)),
            pl.BlockSpec((TN, A), lambda i: (i, 0)),
            pl.BlockSpec((1, A), lambda i: (0, 0)),
            pl.BlockSpec((1, A), lambda i: (0, 0)),
            pl.BlockSpec((1, A), lambda i: (0, 0)),
            pl.BlockSpec((1, A), lambda i: (0, 0)),
            pl.BlockSpec((A, PD), lambda i: (0, 0)),
            pl.BlockSpec((A, PD), lambda i: (0, 0)),
            pl.BlockSpec((1, PD), lambda i: (0, 0)),
        ],
        out_specs=[
            pl.BlockSpec((TN, A), lambda i: (i, 0)),
            pl.BlockSpec((TN, PD), lambda i: (i, 0)),
            pl.BlockSpec((TN, 4 * PD), lambda i: (i, 0)),
        ],
        out_shape=[
            jax.ShapeDtypeStruct((Nn, A), F32),
            jax.ShapeDtypeStruct((Nn, PD), F32),
            jax.ShapeDtypeStruct((Nn, 4 * PD), F32),
        ],
    )(y, atom_fea, a2, c2, ln_atom_g.reshape(1, A), ln_atom_b.reshape(1, A),
      bct, bnt, bu0_b.reshape(1, PD))

    QPAD = NPAD * Mm // 4
    g2p = _sc2_call(pn, idx_r2, QPAD, ST2)

    TN6 = 200
    R6 = TN6 * Mm // 4
    Q = Nn * Mm // 4
    I4 = jnp.eye(4, dtype=F32)
    bbt4 = jnp.kron(I4, bu0_W[:, 2 * A:].T)
    bu1t4 = jnp.kron(I4, bu1_W.T)
    bu2t4 = jnp.kron(I4, bu2_W.T)
    onesb = jnp.kron(I4, jnp.ones((1, B), F32))
    mavg = jnp.kron(I4, jnp.full((B, B), 1.0 / B, F32))
    nbrp = nbr_fea.reshape(Q, 4 * B)
    bg4 = bond_weights_bg_i.reshape(Q, 4)
    nbr_out = pl.pallas_call(
        _k6_body,
        grid=(Nn // TN6,),
        in_specs=[
            pl.BlockSpec((TN6, PD), lambda i: (i, 0)),
            pl.BlockSpec((R6, 128), lambda i: (i, 0)),
            pl.BlockSpec((R6, 4 * B), lambda i: (i, 0)),
            pl.BlockSpec((R6, 4), lambda i: (i, 0)),
            pl.BlockSpec((4 * B, 128), lambda i: (0, 0)),
            pl.BlockSpec((128, 4 * B), lambda i: (0, 0)),
            pl.BlockSpec((4 * B, 4 * B), lambda i: (0, 0)),
            pl.BlockSpec((1, 4 * B), lambda i: (0, 0)),
            pl.BlockSpec((1, 4 * B), lambda i: (0, 0)),
            pl.BlockSpec((4, 4 * B), lambda i: (0, 0)),
            pl.BlockSpec((4 * B, 4 * B), lambda i: (0, 0)),
            pl.BlockSpec((1, 4 * B), lambda i: (0, 0)),
            pl.BlockSpec((1, 4 * B), lambda i: (0, 0)),
        ],
        out_specs=pl.BlockSpec((R6, 4 * B), lambda i: (i, 0)),
        out_shape=jax.ShapeDtypeStruct((Q, 4 * B), F32),
    )(pc, g2p, nbrp, bg4, bbt4, bu1t4, bu2t4,
      jnp.tile(bu1_b, 4).reshape(1, 4 * B), jnp.tile(bu2_b, 4).reshape(1, 4 * B),
      onesb, mavg,
      jnp.tile(ln_bond_g, 4).reshape(1, 4 * B),
      jnp.tile(ln_bond_b, 4).reshape(1, 4 * B))

    return (atom_out, nbr_out.reshape(Nn, Mm, B))


# speedup vs baseline: 2.8077x; 1.4809x over previous
import functools

import jax
import jax.numpy as jnp
from jax import lax
from jax.experimental import pallas as pl
from jax.experimental.pallas import tpu as pltpu
from jax.experimental.pallas import tpu_sc as plsc

F32 = jnp.float32
_EPS = 1e-5

_BCAST_DNUMS = lax.GatherDimensionNumbers(
    offset_dims=(), collapsed_slice_dims=(0,), start_index_map=(0,))


def _bcast16(v, lane):
    idx = jnp.full((16, 1), lane, jnp.int32)
    return lax.gather(v, idx, _BCAST_DNUMS, (1,),
                      indices_are_sorted=True, unique_indices=False,
                      mode=lax.GatherScatterMode.PROMISE_IN_BOUNDS)


def _softplus(x):
    return jnp.maximum(x, 0.0) + jnp.log1p(jnp.exp(-jnp.abs(x)))


def _silu(x):
    return x * jax.nn.sigmoid(x)


def _stage_to_spmem(table_h, shared, buf, sid, chunks, rows):
    for c in range(chunks):
        base = sid * (chunks * rows) + c * rows
        pltpu.sync_copy(table_h.at[pl.ds(base, rows)], buf)
        pltpu.sync_copy(buf, shared.at[pl.ds(base, rows)])
    plsc.subcore_barrier()


def _sc1_call(table, idx_r, w_r, npad, rpw, steps):
    A = table.shape[1]
    NB = 2
    EW = 128
    mesh = plsc.VectorSubcoreMesh(core_axis_name="c", subcore_axis_name="s",
                                  num_cores=1)

    WS = 40

    @functools.partial(
        pl.kernel, mesh=mesh,
        out_type=jax.ShapeDtypeStruct((npad, A), F32),
        scratch_types=(
            [pltpu.VMEM_SHARED((npad, A), F32)]
            + [pltpu.VMEM((WS, EW), jnp.int32)]
            + [pltpu.VMEM((WS, EW), F32)]
            + [pltpu.VMEM((EW, A), F32)] * NB
            + [pltpu.VMEM((EW // 32, A), F32)] * NB
            + [pltpu.SemaphoreType.DMA] * (2 * NB)
        ),
    )
    def sc1(table_h, idx_h, w_h, out_h, shared, idx_v, w_v, *bufs):
        sid = lax.axis_index("s")
        wid = sid
        gbs = bufs[:NB]
        obs = bufs[NB:2 * NB]
        gsems = bufs[2 * NB:3 * NB]
        osems = bufs[3 * NB:4 * NB]
        _stage_to_spmem(table_h, shared, gbs[0], sid, npad // (16 * EW), EW)

        def visit(gt, lt, j):
            @pl.when(gt >= NB)
            def _():
                pltpu.make_async_copy(
                    obs[j], out_h.at[pl.ds(0, EW // 32)], osems[j]).wait()

            pltpu.make_async_copy(
                table_h.at[pl.ds(0, EW)], gbs[j], gsems[j]).wait()
            gbuf, obuf = gbs[j], obs[j]
            for r4 in range(EW // 32):
                acc = [jnp.zeros((16,), F32) for _ in range(A // 16)]
                for g in range(2):
                    wv = w_v[lt, pl.ds((r4 * 2 + g) * 16, 16)]
                    for ln in range(16):
                        wb = _bcast16(wv, ln)
                        e = r4 * 32 + g * 16 + ln
                        for c in range(A // 16):
                            acc[c] = acc[c] + wb * gbuf[e, pl.ds(c * 16, 16)]
                for c in range(A // 16):
                    obuf[r4, pl.ds(c * 16, 16)] = acc[c]
            pltpu.async_copy(
                obuf, out_h.at[pl.ds(wid * rpw + gt * (EW // 32), EW // 32)],
                osems[j])

            @pl.when(lt + NB < WS)
            def _():
                pltpu.async_copy(shared.at[idx_v.at[lt + NB]], gbs[j], gsems[j])

        def window(win, carry):
            pltpu.sync_copy(idx_h.at[wid, pl.ds(win * WS, WS)], idx_v)
            pltpu.sync_copy(w_h.at[wid, pl.ds(win * WS, WS)], w_v)
            for j in range(NB):
                pltpu.async_copy(shared.at[idx_v.at[j]], gbs[j], gsems[j])

            def group(q, c2):
                for j in range(NB):
                    lt = q * NB + j
                    visit(win * WS + lt, lt, j)
                return c2

            lax.fori_loop(0, WS // NB, group, 0)
            return carry

        lax.fori_loop(0, steps // WS, window, 0)
        for j in range(NB):
            pltpu.make_async_copy(
                obs[j], out_h.at[pl.ds(0, EW // 32)], osems[j]).wait()

    return sc1(table, idx_r, w_r)


def _sc2_call(table, idx_r, nrows_pad_q, steps):
    npad, D = table.shape
    epw_q = steps * 32
    NB = 2
    WS = 40
    mesh = plsc.VectorSubcoreMesh(core_axis_name="c", subcore_axis_name="s",
                                  num_cores=1)

    @functools.partial(
        pl.kernel, mesh=mesh,
        out_type=jax.ShapeDtypeStruct((nrows_pad_q, 128), F32),
        scratch_types=(
            [pltpu.VMEM_SHARED((npad, D), F32)]
            + [pltpu.VMEM((WS, 128), jnp.int32)]
            + [pltpu.VMEM((128, D), F32)] * NB
            + [pltpu.VMEM((32, 128), F32)] * NB
            + [pltpu.SemaphoreType.DMA] * (2 * NB)
        ),
    )
    def sc2(table_h, idx_h, out_h, shared, idx_v, *bufs):
        sid = lax.axis_index("s")
        wid = sid
        gbs = bufs[:NB]
        obs = bufs[NB:2 * NB]
        gsems = bufs[2 * NB:3 * NB]
        osems = bufs[3 * NB:4 * NB]
        _stage_to_spmem(table_h, shared, gbs[0], sid, npad // (16 * 128), 128)

        def visit(gt, lt, j):
            @pl.when(gt >= NB)
            def _():
                pltpu.make_async_copy(
                    obs[j], out_h.at[pl.ds(0, 32)], osems[j]).wait()

            pltpu.make_async_copy(
                table_h.at[pl.ds(0, 128)], gbs[j], gsems[j]).wait()
            gbuf, obuf = gbs[j], obs[j]
            for r in range(32):
                for k in range(4):
                    e = r * 4 + k
                    obuf[r, pl.ds(k * 32, 16)] = gbuf[e, pl.ds(0, 16)]
                    obuf[r, pl.ds(k * 32 + 16, 16)] = gbuf[e, pl.ds(16, 16)]
            pltpu.async_copy(
                obuf, out_h.at[pl.ds(wid * epw_q + gt * 32, 32)], osems[j])

            @pl.when(lt + NB < WS)
            def _():
                pltpu.async_copy(shared.at[idx_v.at[lt + NB]], gbs[j], gsems[j])

        def window(win, carry):
            pltpu.sync_copy(idx_h.at[wid, pl.ds(win * WS, WS)], idx_v)
            for j in range(NB):
                pltpu.async_copy(shared.at[idx_v.at[j]], gbs[j], gsems[j])

            def group(q, c2):
                for j in range(NB):
                    lt = q * NB + j
                    visit(win * WS + lt, lt, j)
                return c2

            lax.fori_loop(0, WS // NB, group, 0)
            return carry

        lax.fori_loop(0, steps // WS, window, 0)
        for j in range(NB):
            pltpu.make_async_copy(
                obs[j], out_h.at[pl.ds(0, 32)], osems[j]).wait()

    return sc2(table, idx_r)


def _k1_body(agi, agj, nbrf, wag_o, sumw_o, p3_o):
    w = agi[...] * agj[...]
    wag_o[...] = w
    sumw_o[...] = jnp.sum(w, axis=1, keepdims=True)
    M = w.shape[1]
    B = p3_o.shape[1]
    acc = jnp.zeros(p3_o.shape, F32)
    for m in range(M):
        acc = acc + w[:, m:m + 1] * nbrf[:, B * m:B * (m + 1)]
    p3_o[...] = acc


def _k2_body(af, g1, p3, sw, wct, wnt, wbt, bias, lin_o, s1_o, s2_o):
    x = af[...] * sw[...]
    lin = (jnp.dot(x, wct[...], preferred_element_type=F32)
           + jnp.dot(g1[...], wnt[...], preferred_element_type=F32)
           + jnp.dot(p3[...], wbt[...], preferred_element_type=F32)
           + bias[...])
    lin_o[...] = lin
    s1 = jnp.sum(lin, axis=0, keepdims=True)
    s2 = jnp.sum(lin * lin, axis=0, keepdims=True)

    @pl.when(pl.program_id(0) == 0)
    def _():
        s1_o[...] = s1
        s2_o[...] = s2

    @pl.when(pl.program_id(0) != 0)
    def _():
        s1_o[...] += s1
        s2_o[...] += s2


def _k3_body(lin, a1, c1, y_o, s1_o, s2_o):
    A = y_o.shape[1]
    ln = lin[...] * a1[...] + c1[...]
    y = jax.nn.sigmoid(ln[:, :A]) * _softplus(ln[:, A:])
    y_o[...] = y
    s1 = jnp.sum(y, axis=0, keepdims=True)
    s2 = jnp.sum(y * y, axis=0, keepdims=True)

    @pl.when(pl.program_id(0) == 0)
    def _():
        s1_o[...] = s1
        s2_o[...] = s2

    @pl.when(pl.program_id(0) != 0)
    def _():
        s1_o[...] += s1
        s2_o[...] += s2


def _k4_body(y, af, a2, c2, lng, lnb, bct, bnt, b0, ao_o, pc_o, pn_o):
    z = _softplus(y[...] * a2[...] + c2[...])
    t = z + af[...]
    mu = jnp.mean(t, axis=1, keepdims=True)
    d = t - mu
    var = jnp.mean(d * d, axis=1, keepdims=True)
    ao = d * lax.rsqrt(var + _EPS) * lng[...] + lnb[...]
    ao_o[...] = ao
    pc_o[...] = jnp.dot(ao, bct[...], preferred_element_type=F32) + b0[...]
    pn = jnp.dot(ao, bnt[...], preferred_element_type=F32)
    pn_o[...] = jnp.concatenate([pn, pn, pn, pn], axis=1)


def _k6_body(pc, g2p, nbrp, bg4, bbt4, bu1t4, bu2t4, b14, b24, onesb, mavg,
             lng4, lnb4, out_o):
    TN = pc.shape[0]
    R = g2p.shape[0]
    pc4 = jnp.concatenate([pc[...]] * 4, axis=1)
    pcb = jnp.broadcast_to(pc4[:, None, :], (TN, 8, 128)).reshape(R, 128)
    bp = jnp.dot(nbrp[...], bbt4[...], preferred_element_type=F32)
    h0 = _silu(pcb + g2p[...].astype(F32) + bp)
    h1 = _silu(jnp.dot(h0, bu1t4[...], preferred_element_type=F32) + b14[...])
    h2 = jnp.dot(h1, bu2t4[...], preferred_element_type=F32) + b24[...]
    bgb = jnp.dot(bg4[...], onesb[...], preferred_element_type=F32)
    v = h2 * bgb + nbrp[...]
    mu = jnp.dot(v, mavg[...], preferred_element_type=F32)
    d = v - mu
    var = jnp.dot(d * d, mavg[...], preferred_element_type=F32)
    out_o[...] = d * lax.rsqrt(var + _EPS) * lng4[...] + lnb4[...]


def kernel(atom_fea, nbr_fea, nbr_fea_idx, bond_weights_ag_i,
           bond_weights_ag_j, bond_weights_bg_i, bond_weights_bg_j,
           fc_full_W, fc_full_b, bn1_g, bn1_b, bn2_g, bn2_b,
           ln_atom_g, ln_atom_b, bu0_W, bu0_b, bu1_W, bu1_b, bu2_W, bu2_b,
           ln_bond_g, ln_bond_b):
    Nn, Mm = nbr_fea_idx.shape
    A = atom_fea.shape[-1]
    B = nbr_fea.shape[-1]
    NW = 16
    NPAD = -(-Nn // 2048) * 2048
    RPW = NPAD // NW
    ST1 = RPW * Mm // 128
    ST2 = RPW * Mm // 128

    def to_worker(x, dtype, st, ew):
        xp = jnp.zeros((NPAD, Mm), dtype).at[:Nn].set(x)
        return xp.reshape(NW, st, ew)

    idx_r1 = to_worker(nbr_fea_idx, jnp.int32, ST1, 128)
    idx_r2 = to_worker(nbr_fea_idx, jnp.int32, ST2, 128)

    TN = 400
    grid = (Nn // TN,)
    nbrf2 = nbr_fea.reshape(Nn, Mm * B)
    wag, sumw, p3 = pl.pallas_call(
        _k1_body,
        grid=grid,
        in_specs=[
            pl.BlockSpec((TN, Mm), lambda i: (i, 0)),
            pl.BlockSpec((TN, Mm), lambda i: (i, 0)),
            pl.BlockSpec((TN, Mm * B), lambda i: (i, 0)),
        ],
        out_specs=[
            pl.BlockSpec((TN, Mm), lambda i: (i, 0)),
            pl.BlockSpec((TN, 1), lambda i: (i, 0)),
            pl.BlockSpec((TN, B), lambda i: (i, 0)),
        ],
        out_shape=[
            jax.ShapeDtypeStruct((Nn, Mm), F32),
            jax.ShapeDtypeStruct((Nn, 1), F32),
            jax.ShapeDtypeStruct((Nn, B), F32),
        ],
    )(bond_weights_ag_i, bond_weights_ag_j, nbrf2)

    w_r = to_worker(wag, F32, ST1, 128)
    af_pad = jnp.concatenate(
        [atom_fea, jnp.zeros((NPAD - Nn, A), F32)], axis=0)

    g1 = _sc1_call(af_pad, idx_r1, w_r, NPAD, RPW, ST1)

    C2 = 2 * A
    wct = fc_full_W[:, :A].T
    wnt = fc_full_W[:, A:2 * A].T
    wbt = fc_full_W[:, 2 * A:].T
    lin, l_s1, l_s2 = pl.pallas_call(
        _k2_body,
        grid=grid,
        in_specs=[
            pl.BlockSpec((TN, A), lambda i: (i, 0)),
            pl.BlockSpec((TN, A), lambda i: (i, 0)),
            pl.BlockSpec((TN, B), lambda i: (i, 0)),
            pl.BlockSpec((TN, 1), lambda i: (i, 0)),
            pl.BlockSpec((A, C2), lambda i: (0, 0)),
            pl.BlockSpec((A, C2), lambda i: (0, 0)),
            pl.BlockSpec((B, C2), lambda i: (0, 0)),
            pl.BlockSpec((1, C2), lambda i: (0, 0)),
        ],
        out_specs=[
            pl.BlockSpec((TN, C2), lambda i: (i, 0)),
            pl.BlockSpec((1, C2), lambda i: (0, 0)),
            pl.BlockSpec((1, C2), lambda i: (0, 0)),
        ],
        out_shape=[
            jax.ShapeDtypeStruct((Nn, C2), F32),
            jax.ShapeDtypeStruct((1, C2), F32),
            jax.ShapeDtypeStruct((1, C2), F32),
        ],
    )(atom_fea, g1, p3, sumw, wct, wnt, wbt, fc_full_b.reshape(1, C2))

    mu1 = l_s1 / Nn
    var1 = l_s2 / Nn - mu1 * mu1
    a1 = bn1_g.reshape(1, C2) * lax.rsqrt(var1 + _EPS)
    c1 = bn1_b.reshape(1, C2) - mu1 * a1

    y, y_s1, y_s2 = pl.pallas_call(
        _k3_body,
        grid=grid,
        in_specs=[
            pl.BlockSpec((TN, C2), lambda i: (i, 0)),
            pl.BlockSpec((1, C2), lambda i: (0, 0)),
            pl.BlockSpec((1, C2), lambda i: (0, 0)),
        ],
        out_specs=[
            pl.BlockSpec((TN, A), lambda i: (i, 0)),
            pl.BlockSpec((1, A), lambda i: (0, 0)),
            pl.BlockSpec((1, A), lambda i: (0, 0)),
        ],
        out_shape=[
            jax.ShapeDtypeStruct((Nn, A), F32),
            jax.ShapeDtypeStruct((1, A), F32),
            jax.ShapeDtypeStruct((1, A), F32),
        ],
    )(lin, a1, c1)

    mu2 = y_s1 / Nn
    var2 = y_s2 / Nn - mu2 * mu2
    a2 = bn2_g.reshape(1, A) * lax.rsqrt(var2 + _EPS)
    c2 = bn2_b.reshape(1, A) - mu2 * a2

    PD = bu0_W.shape[0]
    bct = bu0_W[:, :A].T
    bnt = bu0_W[:, A:2 * A].T
    atom_out, pc, pn = pl.pallas_call(
        _k4_body,
        grid=grid,
        in_specs=[
            pl.BlockSpec((TN, A), lambda i: (i, 0)),
            pl.BlockSpec((TN, A), lambda i: (i, 0)),
            pl.BlockSpec((1, A), lambda i: (0, 0)),
            pl.BlockSpec((1, A), lambda i: (0, 0)),
            pl.BlockSpec((1, A), lambda i: (0, 0)),
            pl.BlockSpec((1, A), lambda i: (0, 0)),
            pl.BlockSpec((A, PD), lambda i: (0, 0)),
            pl.BlockSpec((A, PD), lambda i: (0, 0)),
            pl.BlockSpec((1, PD), lambda i: (0, 0)),
        ],
        out_specs=[
            pl.BlockSpec((TN, A), lambda i: (i, 0)),
            pl.BlockSpec((TN, PD), lambda i: (i, 0)),
            pl.BlockSpec((TN, 4 * PD), lambda i: (i, 0)),
        ],
        out_shape=[
            jax.ShapeDtypeStruct((Nn, A), F32),
            jax.ShapeDtypeStruct((Nn, PD), F32),
            jax.ShapeDtypeStruct((Nn, 4 * PD), F32),
        ],
    )(y, atom_fea, a2, c2, ln_atom_g.reshape(1, A), ln_atom_b.reshape(1, A),
      bct, bnt, bu0_b.reshape(1, PD))

    QPAD = NPAD * Mm // 4
    pn_pad = jnp.concatenate(
        [pn, jnp.zeros((NPAD - Nn, 4 * PD), F32)], axis=0)
    g2p = _sc2_call(pn_pad, idx_r2, QPAD, ST2)

    TN6 = 200
    R6 = TN6 * Mm // 4
    Q = Nn * Mm // 4
    I4 = jnp.eye(4, dtype=F32)
    bbt4 = jnp.kron(I4, bu0_W[:, 2 * A:].T)
    bu1t4 = jnp.kron(I4, bu1_W.T)
    bu2t4 = jnp.kron(I4, bu2_W.T)
    onesb = jnp.kron(I4, jnp.ones((1, B), F32))
    mavg = jnp.kron(I4, jnp.full((B, B), 1.0 / B, F32))
    nbrp = nbr_fea.reshape(Q, 4 * B)
    bg4 = bond_weights_bg_i.reshape(Q, 4)
    nbr_out = pl.pallas_call(
        _k6_body,
        grid=(Nn // TN6,),
        in_specs=[
            pl.BlockSpec((TN6, PD), lambda i: (i, 0)),
            pl.BlockSpec((R6, 128), lambda i: (i, 0)),
            pl.BlockSpec((R6, 4 * B), lambda i: (i, 0)),
            pl.BlockSpec((R6, 4), lambda i: (i, 0)),
            pl.BlockSpec((4 * B, 128), lambda i: (0, 0)),
            pl.BlockSpec((128, 4 * B), lambda i: (0, 0)),
            pl.BlockSpec((4 * B, 4 * B), lambda i: (0, 0)),
            pl.BlockSpec((1, 4 * B), lambda i: (0, 0)),
            pl.BlockSpec((1, 4 * B), lambda i: (0, 0)),
            pl.BlockSpec((4, 4 * B), lambda i: (0, 0)),
            pl.BlockSpec((4 * B, 4 * B), lambda i: (0, 0)),
            pl.BlockSpec((1, 4 * B), lambda i: (0, 0)),
            pl.BlockSpec((1, 4 * B), lambda i: (0, 0)),
        ],
        out_specs=pl.BlockSpec((R6, 4 * B), lambda i: (i, 0)),
        out_shape=jax.ShapeDtypeStruct((Q, 4 * B), F32),
    )(pc, g2p, nbrp, bg4, bbt4, bu1t4, bu2t4,
      jnp.tile(bu1_b, 4).reshape(1, 4 * B), jnp.tile(bu2_b, 4).reshape(1, 4 * B),
      onesb, mavg,
      jnp.tile(ln_bond_g, 4).reshape(1, 4 * B),
      jnp.tile(ln_bond_b, 4).reshape(1, 4 * B))

    return (atom_out, nbr_out.reshape(Nn, Mm, B))
